# Initial kernel scaffold; baseline (speedup 1.0000x reference)
import functools

import jax
import jax.numpy as jnp
from jax import lax
from jax.experimental import pallas as pl
from jax.experimental.pallas import tpu as pltpu
from jax.experimental.pallas import tpu_sc as plsc

_N = 10000
_E = 320000
_R = 10240
_NW = 32
_CHUNK = 128
_CPT = 79
_EPT = _E // _NW
_PAD = _CPT * _CHUNK - _EPT
_BLK = 1280
_GRID = _R // _BLK


def _sc_mesh():
    return plsc.VectorSubcoreMesh(core_axis_name="c", subcore_axis_name="s")


def _sc_degree(dstp):
    @functools.partial(
        pl.kernel,
        out_type=jax.ShapeDtypeStruct((2, _R), jnp.float32),
        mesh=_sc_mesh(),
        scratch_types=[
            pltpu.VMEM((_CPT, _CHUNK), jnp.int32),
            pltpu.VMEM((_CHUNK,), jnp.float32),
            pltpu.VMEM((640,), jnp.float32),
            pltpu.VMEM_SHARED((_R,), jnp.float32),
        ],
    )
    def deg_kernel(dstp_hbm, out_hbm, idx_v, ones_v, zb_v, acc):
        c = lax.axis_index("c")
        s = lax.axis_index("s")
        wid = c * 16 + s
        zeros16 = jnp.zeros((16,), jnp.float32)
        ones16 = jnp.ones((16,), jnp.float32)
        for k in range(40):
            zb_v[pl.ds(k * 16, 16)] = zeros16
        for k in range(8):
            ones_v[pl.ds(k * 16, 16)] = ones16
        pltpu.sync_copy(zb_v, acc.at[pl.ds(s * 640, 640)])
        pltpu.sync_copy(dstp_hbm.at[wid], idx_v)
        plsc.subcore_barrier()

        def body(j, carry):
            pltpu.sync_copy(ones_v, acc.at[idx_v.at[j]], add=True)
            return carry

        lax.fori_loop(0, _CPT, body, 0)
        plsc.subcore_barrier()
        pltpu.sync_copy(acc.at[pl.ds(s * 640, 640)],
                        out_hbm.at[c, pl.ds(s * 640, 640)])

    return deg_kernel(dstp)


def _sc_aggregate(table, srcp, dstp, d):
    @functools.partial(
        pl.kernel,
        out_type=jax.ShapeDtypeStruct((2, _R, d), jnp.float32),
        mesh=_sc_mesh(),
        scratch_types=[
            pltpu.VMEM((_CPT, _CHUNK), jnp.int32),
            pltpu.VMEM((_CPT, _CHUNK), jnp.int32),
            pltpu.VMEM((_CHUNK, d), jnp.float32),
            pltpu.VMEM((16, d), jnp.float32),
            pltpu.VMEM_SHARED((_R, d), jnp.float32),
            pltpu.SemaphoreType.DMA,
        ],
    )
    def agg_kernel(table_hbm, srcp_hbm, dstp_hbm, out_hbm,
                   sidx, didx, rows, zb, acc, sem):
        c = lax.axis_index("c")
        s = lax.axis_index("s")
        wid = c * 16 + s
        zeros16 = jnp.zeros((16,), jnp.float32)
        for r in range(16):
            for q in range(d // 16):
                zb[r, pl.ds(q * 16, 16)] = zeros16
        for k in range(40):
            pltpu.sync_copy(zb, acc.at[pl.ds(s * 640 + k * 16, 16)])
        pltpu.sync_copy(srcp_hbm.at[wid], sidx)
        pltpu.sync_copy(dstp_hbm.at[wid], didx)
        plsc.subcore_barrier()

        def body(j, carry):
            pltpu.async_copy(table_hbm.at[sidx.at[j]], rows, sem).wait()
            pltpu.sync_copy(rows, acc.at[didx.at[j]], add=True)
            return carry

        lax.fori_loop(0, _CPT, body, 0)
        plsc.subcore_barrier()
        pltpu.sync_copy(acc.at[pl.ds(s * 640, 640)],
                        out_hbm.at[c, pl.ds(s * 640, 640)])

    return agg_kernel(table, srcp, dstp)


def _tc1_body(degT_ref, x_ref, w_ref, out_ref):
    deg = degT_ref[:, 0:1] + degT_ref[:, 1:2] + 1.0
    dinv = lax.rsqrt(deg)
    h = jnp.dot(x_ref[...], w_ref[...], preferred_element_type=jnp.float32)
    out_ref[...] = h * dinv


def _tc_scale_matmul(degT, xp, w):
    return pl.pallas_call(
        _tc1_body,
        grid=(_GRID,),
        in_specs=[
            pl.BlockSpec((_BLK, 2), lambda i: (i, 0)),
            pl.BlockSpec((_BLK, 128), lambda i: (i, 0)),
            pl.BlockSpec((128, 128), lambda i: (0, 0)),
        ],
        out_specs=pl.BlockSpec((_BLK, 128), lambda i: (i, 0)),
        out_shape=jax.ShapeDtypeStruct((_R, 128), jnp.float32),
    )(degT, xp, w)


def _tc2_body(sacc_ref, h1_ref, degT_ref, b1_ref, w2_ref, out_ref):
    deg = degT_ref[:, 0:1] + degT_ref[:, 1:2] + 1.0
    dinv = lax.rsqrt(deg)
    s = sacc_ref[0] + sacc_ref[1] + h1_ref[...]
    z = jnp.maximum(s * dinv + b1_ref[...], 0.0)
    h2 = jnp.dot(z, w2_ref[...], preferred_element_type=jnp.float32)
    out_ref[...] = h2 * dinv


def _tc_mid(sacc, h1, degT, b1, w2):
    return pl.pallas_call(
        _tc2_body,
        grid=(_GRID,),
        in_specs=[
            pl.BlockSpec((2, _BLK, 128), lambda i: (0, i, 0)),
            pl.BlockSpec((_BLK, 128), lambda i: (i, 0)),
            pl.BlockSpec((_BLK, 2), lambda i: (i, 0)),
            pl.BlockSpec((1, 128), lambda i: (0, 0)),
            pl.BlockSpec((128, 64), lambda i: (0, 0)),
        ],
        out_specs=pl.BlockSpec((_BLK, 64), lambda i: (i, 0)),
        out_shape=jax.ShapeDtypeStruct((_R, 64), jnp.float32),
    )(sacc, h1, degT, b1, w2)


def _tc3_body(sacc_ref, h2_ref, degT_ref, b2_ref, out_ref):
    deg = degT_ref[:, 0:1] + degT_ref[:, 1:2] + 1.0
    dinv = lax.rsqrt(deg)
    s = sacc_ref[0] + sacc_ref[1] + h2_ref[...]
    out_ref[...] = s * dinv + b2_ref[...]


def _tc_final(sacc, h2, degT, b2):
    return pl.pallas_call(
        _tc3_body,
        grid=(_GRID,),
        in_specs=[
            pl.BlockSpec((2, _BLK, 64), lambda i: (0, i, 0)),
            pl.BlockSpec((_BLK, 64), lambda i: (i, 0)),
            pl.BlockSpec((_BLK, 2), lambda i: (i, 0)),
            pl.BlockSpec((1, 64), lambda i: (0, 0)),
        ],
        out_specs=pl.BlockSpec((_BLK, 64), lambda i: (i, 0)),
        out_shape=jax.ShapeDtypeStruct((_R, 64), jnp.float32),
    )(sacc, h2, degT, b2)


def kernel(x, edge_index, W1, b1, W2, b2):
    src = edge_index[0]
    dst = edge_index[1]

    w = jnp.arange(_NW, dtype=jnp.int32)[:, None]
    i = jnp.arange(_PAD, dtype=jnp.int32)[None, :]
    pad_src = (w * 997 + i * 131) % _N
    pad_dst = _N + (w + i) % 16
    srcp = jnp.concatenate([src.reshape(_NW, _EPT), pad_src], axis=1)
    dstp = jnp.concatenate([dst.reshape(_NW, _EPT), pad_dst], axis=1)
    srcp = srcp.reshape(_NW, _CPT, _CHUNK)
    dstp = dstp.reshape(_NW, _CPT, _CHUNK)

    degp = _sc_degree(dstp)
    degT = degp.T

    xp = jnp.concatenate(
        [x, jnp.zeros((_R - _N, x.shape[1]), x.dtype)], axis=0)

    h1 = _tc_scale_matmul(degT, xp, W1)
    s1 = _sc_aggregate(h1, srcp, dstp, 128)
    h2 = _tc_mid(s1, h1, degT, b1.reshape(1, -1), W2)
    s2 = _sc_aggregate(h2, srcp, dstp, 64)
    outp = _tc_final(s2, h2, degT, b2.reshape(1, -1))
    return outp[:_N]


# speedup vs baseline: 25.3537x; 25.3537x over previous
import functools

import jax
import jax.numpy as jnp
from jax import lax
from jax.experimental import pallas as pl
from jax.experimental.pallas import tpu as pltpu
from jax.experimental.pallas import tpu_sc as plsc

_N = 10000
_E = 320000
_R = 10240
_NW = 32
_CHUNK = 128
_CPT = 79
_EPT = _E // _NW
_PAD = _CPT * _CHUNK - _EPT
_BLK = 1280
_GRID = _R // _BLK


def _sc_mesh():
    return plsc.VectorSubcoreMesh(core_axis_name="c", subcore_axis_name="s")


def _sc_degree(dstp):
    @functools.partial(
        pl.kernel,
        out_type=jax.ShapeDtypeStruct((2, _R), jnp.float32),
        mesh=_sc_mesh(),
        scratch_types=[
            pltpu.VMEM((_CPT, _CHUNK), jnp.int32),
            pltpu.VMEM((_CHUNK,), jnp.float32),
            pltpu.VMEM((640,), jnp.float32),
            pltpu.VMEM_SHARED((_R,), jnp.float32),
        ],
    )
    def deg_kernel(dstp_hbm, out_hbm, idx_v, ones_v, zb_v, acc):
        c = lax.axis_index("c")
        s = lax.axis_index("s")
        wid = c * 16 + s
        zeros16 = jnp.zeros((16,), jnp.float32)
        ones16 = jnp.ones((16,), jnp.float32)
        for k in range(40):
            zb_v[pl.ds(k * 16, 16)] = zeros16
        for k in range(8):
            ones_v[pl.ds(k * 16, 16)] = ones16
        pltpu.sync_copy(zb_v, acc.at[pl.ds(s * 640, 640)])
        pltpu.sync_copy(dstp_hbm.at[wid], idx_v)
        plsc.subcore_barrier()

        def body(j, carry):
            pltpu.sync_copy(ones_v, acc.at[idx_v.at[j]], add=True)
            return carry

        lax.fori_loop(0, _CPT, body, 0)
        plsc.subcore_barrier()
        pltpu.sync_copy(acc.at[pl.ds(s * 640, 640)],
                        out_hbm.at[c, pl.ds(s * 640, 640)])

    return deg_kernel(dstp)


def _sc_aggregate(table, srcp, dstp, d):
    @functools.partial(
        pl.kernel,
        out_type=jax.ShapeDtypeStruct((2, _R, d), jnp.float32),
        mesh=_sc_mesh(),
        compiler_params=pltpu.CompilerParams(use_tc_tiling_on_sc=False),
        scratch_types=[
            pltpu.VMEM((_CPT, _CHUNK), jnp.int32),
            pltpu.VMEM((_CPT, _CHUNK), jnp.int32),
            pltpu.VMEM((_CHUNK, d), jnp.float32),
            pltpu.VMEM((16, d), jnp.float32),
            pltpu.VMEM_SHARED((_R, d), jnp.float32),
            pltpu.SemaphoreType.DMA,
        ],
    )
    def agg_kernel(table_hbm, srcp_hbm, dstp_hbm, out_hbm,
                   sidx, didx, rows, zb, acc, sem):
        c = lax.axis_index("c")
        s = lax.axis_index("s")
        wid = c * 16 + s
        zeros16 = jnp.zeros((16,), jnp.float32)
        for r in range(16):
            for q in range(d // 16):
                zb[r, pl.ds(q * 16, 16)] = zeros16
        for k in range(40):
            pltpu.sync_copy(zb, acc.at[pl.ds(s * 640 + k * 16, 16)])
        pltpu.sync_copy(srcp_hbm.at[wid], sidx)
        pltpu.sync_copy(dstp_hbm.at[wid], didx)
        plsc.subcore_barrier()

        def body(j, carry):
            pltpu.async_copy(table_hbm.at[sidx.at[j]], rows, sem).wait()
            pltpu.sync_copy(rows, acc.at[didx.at[j]], add=True)
            return carry

        lax.fori_loop(0, _CPT, body, 0)
        plsc.subcore_barrier()
        pltpu.sync_copy(acc.at[pl.ds(s * 640, 640)],
                        out_hbm.at[c, pl.ds(s * 640, 640)])

    return agg_kernel(table, srcp, dstp)


def _tc1_body(degT_ref, x_ref, w_ref, out_ref):
    deg = degT_ref[:, 0:1] + degT_ref[:, 1:2] + 1.0
    dinv = lax.rsqrt(deg)
    h = jnp.dot(x_ref[...], w_ref[...], preferred_element_type=jnp.float32)
    out_ref[...] = h * dinv


def _tc_scale_matmul(degT, xp, w):
    return pl.pallas_call(
        _tc1_body,
        grid=(_GRID,),
        in_specs=[
            pl.BlockSpec((_BLK, 2), lambda i: (i, 0)),
            pl.BlockSpec((_BLK, 128), lambda i: (i, 0)),
            pl.BlockSpec((128, 128), lambda i: (0, 0)),
        ],
        out_specs=pl.BlockSpec((_BLK, 128), lambda i: (i, 0)),
        out_shape=jax.ShapeDtypeStruct((_R, 128), jnp.float32),
    )(degT, xp, w)


def _tc2_body(sacc_ref, h1_ref, degT_ref, b1_ref, w2_ref, out_ref):
    deg = degT_ref[:, 0:1] + degT_ref[:, 1:2] + 1.0
    dinv = lax.rsqrt(deg)
    s = sacc_ref[0] + sacc_ref[1] + h1_ref[...]
    z = jnp.maximum(s * dinv + b1_ref[...], 0.0)
    h2 = jnp.dot(z, w2_ref[...], preferred_element_type=jnp.float32)
    out_ref[...] = h2 * dinv


def _tc_mid(sacc, h1, degT, b1, w2):
    return pl.pallas_call(
        _tc2_body,
        grid=(_GRID,),
        in_specs=[
            pl.BlockSpec((2, _BLK, 128), lambda i: (0, i, 0)),
            pl.BlockSpec((_BLK, 128), lambda i: (i, 0)),
            pl.BlockSpec((_BLK, 2), lambda i: (i, 0)),
            pl.BlockSpec((1, 128), lambda i: (0, 0)),
            pl.BlockSpec((128, 64), lambda i: (0, 0)),
        ],
        out_specs=pl.BlockSpec((_BLK, 64), lambda i: (i, 0)),
        out_shape=jax.ShapeDtypeStruct((_R, 64), jnp.float32),
    )(sacc, h1, degT, b1, w2)


def _tc3_body(sacc_ref, h2_ref, degT_ref, b2_ref, out_ref):
    deg = degT_ref[:, 0:1] + degT_ref[:, 1:2] + 1.0
    dinv = lax.rsqrt(deg)
    s = sacc_ref[0] + sacc_ref[1] + h2_ref[...]
    out_ref[...] = s * dinv + b2_ref[...]


def _tc_final(sacc, h2, degT, b2):
    return pl.pallas_call(
        _tc3_body,
        grid=(_GRID,),
        in_specs=[
            pl.BlockSpec((2, _BLK, 64), lambda i: (0, i, 0)),
            pl.BlockSpec((_BLK, 64), lambda i: (i, 0)),
            pl.BlockSpec((_BLK, 2), lambda i: (i, 0)),
            pl.BlockSpec((1, 64), lambda i: (0, 0)),
        ],
        out_specs=pl.BlockSpec((_BLK, 64), lambda i: (i, 0)),
        out_shape=jax.ShapeDtypeStruct((_R, 64), jnp.float32),
    )(sacc, h2, degT, b2)


def kernel(x, edge_index, W1, b1, W2, b2):
    src = edge_index[0]
    dst = edge_index[1]

    w = jnp.arange(_NW, dtype=jnp.int32)[:, None]
    i = jnp.arange(_PAD, dtype=jnp.int32)[None, :]
    pad_src = (w * 997 + i * 131) % _N
    pad_dst = _N + (w + i) % 16
    srcp = jnp.concatenate([src.reshape(_NW, _EPT), pad_src], axis=1)
    dstp = jnp.concatenate([dst.reshape(_NW, _EPT), pad_dst], axis=1)
    srcp = srcp.reshape(_NW, _CPT, _CHUNK)
    dstp = dstp.reshape(_NW, _CPT, _CHUNK)

    degp = _sc_degree(dstp)
    degT = degp.T

    xp = jnp.concatenate(
        [x, jnp.zeros((_R - _N, x.shape[1]), x.dtype)], axis=0)

    h1 = _tc_scale_matmul(degT, xp, W1)
    s1 = _sc_aggregate(h1, srcp, dstp, 128)
    h2 = _tc_mid(s1, h1, degT, b1.reshape(1, -1), W2)
    s2 = _sc_aggregate(h2, srcp, dstp, 64)
    outp = _tc_final(s2, h2, degT, b2.reshape(1, -1))
    return outp[:_N]


# speedup vs baseline: 28.5480x; 1.1260x over previous
import functools

import jax
import jax.numpy as jnp
from jax import lax
from jax.experimental import pallas as pl
from jax.experimental.pallas import tpu as pltpu
from jax.experimental.pallas import tpu_sc as plsc

_N = 10000
_E = 320000
_R = 10240
_NW = 32
_CHUNK = 128
_CPT = 80
_GRP = 8
_EPT = _E // _NW
_PAD = _CPT * _CHUNK - _EPT
_BLK = 1280
_GRID = _R // _BLK


def _sc_mesh():
    return plsc.VectorSubcoreMesh(core_axis_name="c", subcore_axis_name="s")


def _sc_degree(dstp):
    @functools.partial(
        pl.kernel,
        out_type=jax.ShapeDtypeStruct((2, _R), jnp.float32),
        mesh=_sc_mesh(),
        scratch_types=[
            pltpu.VMEM((_CPT, _CHUNK), jnp.int32),
            pltpu.VMEM((_CHUNK,), jnp.float32),
            pltpu.VMEM((640,), jnp.float32),
            pltpu.VMEM_SHARED((_R,), jnp.float32),
        ],
    )
    def deg_kernel(dstp_hbm, out_hbm, idx_v, ones_v, zb_v, acc):
        c = lax.axis_index("c")
        s = lax.axis_index("s")
        wid = c * 16 + s
        zeros16 = jnp.zeros((16,), jnp.float32)
        ones16 = jnp.ones((16,), jnp.float32)
        for k in range(40):
            zb_v[pl.ds(k * 16, 16)] = zeros16
        for k in range(8):
            ones_v[pl.ds(k * 16, 16)] = ones16
        pltpu.sync_copy(zb_v, acc.at[pl.ds(s * 640, 640)])
        pltpu.sync_copy(dstp_hbm.at[wid], idx_v)
        plsc.subcore_barrier()

        def body(j, carry):
            pltpu.sync_copy(ones_v, acc.at[idx_v.at[j]], add=True)
            return carry

        lax.fori_loop(0, _CPT, body, 0)
        plsc.subcore_barrier()
        pltpu.sync_copy(acc.at[pl.ds(s * 640, 640)],
                        out_hbm.at[c, pl.ds(s * 640, 640)])

    return deg_kernel(dstp)


def _sc_aggregate(table, srcp, dstp, d):
    @functools.partial(
        pl.kernel,
        out_type=jax.ShapeDtypeStruct((2, _R, d), jnp.float32),
        mesh=_sc_mesh(),
        compiler_params=pltpu.CompilerParams(use_tc_tiling_on_sc=False),
        scratch_types=[
            pltpu.VMEM((_GRP, _CHUNK), jnp.int32),
            pltpu.VMEM((_GRP, _CHUNK), jnp.int32),
            pltpu.VMEM((_CHUNK, d), jnp.float32),
            pltpu.VMEM((_CHUNK, d), jnp.float32),
            pltpu.VMEM((16, d), jnp.float32),
            pltpu.VMEM_SHARED((_R, d), jnp.float32),
            pltpu.SemaphoreType.DMA,
            pltpu.SemaphoreType.DMA,
        ],
    )
    def agg_kernel(table_hbm, srcp_hbm, dstp_hbm, out_hbm,
                   sidx, didx, rows0, rows1, zb, acc, sem0, sem1):
        c = lax.axis_index("c")
        s = lax.axis_index("s")
        wid = c * 16 + s
        zeros16 = jnp.zeros((16,), jnp.float32)
        for r in range(16):
            for q in range(d // 16):
                zb[r, pl.ds(q * 16, 16)] = zeros16
        for k in range(40):
            pltpu.sync_copy(zb, acc.at[pl.ds(s * 640 + k * 16, 16)])
        plsc.subcore_barrier()

        bufs = ((rows0, sem0), (rows1, sem1))

        def group(g, carry):
            pltpu.sync_copy(srcp_hbm.at[wid, pl.ds(g * _GRP, _GRP)], sidx)
            pltpu.sync_copy(dstp_hbm.at[wid, pl.ds(g * _GRP, _GRP)], didx)
            pltpu.async_copy(table_hbm.at[sidx.at[0]], rows0, sem0)
            for j in range(_GRP):
                buf, sem = bufs[j % 2]
                pltpu.make_async_copy(table_hbm.at[sidx.at[j]], buf,
                                      sem).wait()
                if j + 1 < _GRP:
                    nbuf, nsem = bufs[(j + 1) % 2]
                    pltpu.async_copy(table_hbm.at[sidx.at[j + 1]], nbuf,
                                     nsem)
                pltpu.sync_copy(buf, acc.at[didx.at[j]], add=True)
            return carry

        lax.fori_loop(0, _CPT // _GRP, group, 0)
        plsc.subcore_barrier()
        pltpu.sync_copy(acc.at[pl.ds(s * 640, 640)],
                        out_hbm.at[c, pl.ds(s * 640, 640)])

    return agg_kernel(table, srcp, dstp)


def _tc1_body(degT_ref, x_ref, w_ref, out_ref):
    deg = degT_ref[:, 0:1] + degT_ref[:, 1:2] + 1.0
    dinv = lax.rsqrt(deg)
    h = jnp.dot(x_ref[...], w_ref[...], preferred_element_type=jnp.float32)
    out_ref[...] = h * dinv


def _tc_scale_matmul(degT, xp, w):
    return pl.pallas_call(
        _tc1_body,
        grid=(_GRID,),
        in_specs=[
            pl.BlockSpec((_BLK, 2), lambda i: (i, 0)),
            pl.BlockSpec((_BLK, 128), lambda i: (i, 0)),
            pl.BlockSpec((128, 128), lambda i: (0, 0)),
        ],
        out_specs=pl.BlockSpec((_BLK, 128), lambda i: (i, 0)),
        out_shape=jax.ShapeDtypeStruct((_R, 128), jnp.float32),
    )(degT, xp, w)


def _tc2_body(sacc_ref, h1_ref, degT_ref, b1_ref, w2_ref, out_ref):
    deg = degT_ref[:, 0:1] + degT_ref[:, 1:2] + 1.0
    dinv = lax.rsqrt(deg)
    s = sacc_ref[0] + sacc_ref[1] + h1_ref[...]
    z = jnp.maximum(s * dinv + b1_ref[...], 0.0)
    h2 = jnp.dot(z, w2_ref[...], preferred_element_type=jnp.float32)
    out_ref[...] = h2 * dinv


def _tc_mid(sacc, h1, degT, b1, w2):
    return pl.pallas_call(
        _tc2_body,
        grid=(_GRID,),
        in_specs=[
            pl.BlockSpec((2, _BLK, 128), lambda i: (0, i, 0)),
            pl.BlockSpec((_BLK, 128), lambda i: (i, 0)),
            pl.BlockSpec((_BLK, 2), lambda i: (i, 0)),
            pl.BlockSpec((1, 128), lambda i: (0, 0)),
            pl.BlockSpec((128, 64), lambda i: (0, 0)),
        ],
        out_specs=pl.BlockSpec((_BLK, 64), lambda i: (i, 0)),
        out_shape=jax.ShapeDtypeStruct((_R, 64), jnp.float32),
    )(sacc, h1, degT, b1, w2)


def _tc3_body(sacc_ref, h2_ref, degT_ref, b2_ref, out_ref):
    deg = degT_ref[:, 0:1] + degT_ref[:, 1:2] + 1.0
    dinv = lax.rsqrt(deg)
    s = sacc_ref[0] + sacc_ref[1] + h2_ref[...]
    out_ref[...] = s * dinv + b2_ref[...]


def _tc_final(sacc, h2, degT, b2):
    return pl.pallas_call(
        _tc3_body,
        grid=(_GRID,),
        in_specs=[
            pl.BlockSpec((2, _BLK, 64), lambda i: (0, i, 0)),
            pl.BlockSpec((_BLK, 64), lambda i: (i, 0)),
            pl.BlockSpec((_BLK, 2), lambda i: (i, 0)),
            pl.BlockSpec((1, 64), lambda i: (0, 0)),
        ],
        out_specs=pl.BlockSpec((_BLK, 64), lambda i: (i, 0)),
        out_shape=jax.ShapeDtypeStruct((_R, 64), jnp.float32),
    )(sacc, h2, degT, b2)


def kernel(x, edge_index, W1, b1, W2, b2):
    src = edge_index[0]
    dst = edge_index[1]

    w = jnp.arange(_NW, dtype=jnp.int32)[:, None]
    i = jnp.arange(_PAD, dtype=jnp.int32)[None, :]
    pad_src = (w * 997 + i * 131) % _N
    pad_dst = _N + (w + i) % 16
    srcp = jnp.concatenate([src.reshape(_NW, _EPT), pad_src], axis=1)
    dstp = jnp.concatenate([dst.reshape(_NW, _EPT), pad_dst], axis=1)
    srcp = srcp.reshape(_NW, _CPT, _CHUNK)
    dstp = dstp.reshape(_NW, _CPT, _CHUNK)

    degp = _sc_degree(dstp)
    degT = degp.T

    xp = jnp.concatenate(
        [x, jnp.zeros((_R - _N, x.shape[1]), x.dtype)], axis=0)

    h1 = _tc_scale_matmul(degT, xp, W1)
    s1 = _sc_aggregate(h1, srcp, dstp, 128)
    h2 = _tc_mid(s1, h1, degT, b1.reshape(1, -1), W2)
    s2 = _sc_aggregate(h2, srcp, dstp, 64)
    outp = _tc_final(s2, h2, degT, b2.reshape(1, -1))
    return outp[:_N]


# speedup vs baseline: 30.2579x; 1.0599x over previous
import functools

import jax
import jax.numpy as jnp
from jax import lax
from jax.experimental import pallas as pl
from jax.experimental.pallas import tpu as pltpu
from jax.experimental.pallas import tpu_sc as plsc

_N = 10000
_E = 320000
_R = 10240
_NW = 32
_CHUNK = 128
_CPT = 80
_GRP = 8
_EPT = _E // _NW
_PAD = _CPT * _CHUNK - _EPT
_BLK = 1280
_GRID = _R // _BLK


def _sc_mesh():
    return plsc.VectorSubcoreMesh(core_axis_name="c", subcore_axis_name="s")


def _sc_degree(dstp):
    @functools.partial(
        pl.kernel,
        out_type=jax.ShapeDtypeStruct((2, _R), jnp.float32),
        mesh=_sc_mesh(),
        scratch_types=[
            pltpu.VMEM((_CPT, _CHUNK), jnp.int32),
            pltpu.VMEM((_CHUNK,), jnp.float32),
            pltpu.VMEM((640,), jnp.float32),
            pltpu.VMEM_SHARED((_R,), jnp.float32),
        ],
    )
    def deg_kernel(dstp_hbm, out_hbm, idx_v, ones_v, zb_v, acc):
        c = lax.axis_index("c")
        s = lax.axis_index("s")
        wid = c * 16 + s
        zeros16 = jnp.zeros((16,), jnp.float32)
        ones16 = jnp.ones((16,), jnp.float32)
        for k in range(40):
            zb_v[pl.ds(k * 16, 16)] = zeros16
        for k in range(8):
            ones_v[pl.ds(k * 16, 16)] = ones16
        pltpu.sync_copy(zb_v, acc.at[pl.ds(s * 640, 640)])
        pltpu.sync_copy(dstp_hbm.at[wid], idx_v)
        plsc.subcore_barrier()

        def body(j, carry):
            pltpu.sync_copy(ones_v, acc.at[idx_v.at[j]], add=True)
            return carry

        lax.fori_loop(0, _CPT, body, 0)
        plsc.subcore_barrier()
        pltpu.sync_copy(acc.at[pl.ds(s * 640, 640)],
                        out_hbm.at[c, pl.ds(s * 640, 640)])

    return deg_kernel(dstp)


def _sc_aggregate(table, srcp, dstp, d):
    @functools.partial(
        pl.kernel,
        out_type=jax.ShapeDtypeStruct((2, _R, d), jnp.float32),
        mesh=_sc_mesh(),
        compiler_params=pltpu.CompilerParams(use_tc_tiling_on_sc=False),
        scratch_types=[
            pltpu.VMEM((2, _GRP, _CHUNK), jnp.int32),
            pltpu.VMEM((2, _GRP, _CHUNK), jnp.int32),
            pltpu.VMEM((_CHUNK, d), jnp.float32),
            pltpu.VMEM((_CHUNK, d), jnp.float32),
            pltpu.VMEM((16, d), jnp.float32),
            pltpu.VMEM_SHARED((_R, d), jnp.float32),
            pltpu.SemaphoreType.DMA,
            pltpu.SemaphoreType.DMA,
            pltpu.SemaphoreType.DMA,
            pltpu.SemaphoreType.DMA,
            pltpu.SemaphoreType.DMA,
        ],
    )
    def agg_kernel(table_hbm, srcp_hbm, dstp_hbm, out_hbm,
                   sidx, didx, rows0, rows1, zb, acc,
                   gs0, gs1, ss0, ss1, isem):
        c = lax.axis_index("c")
        s = lax.axis_index("s")
        wid = c * 16 + s
        n_grp = _CPT // _GRP
        zeros16 = jnp.zeros((16,), jnp.float32)
        for r in range(16):
            for q in range(d // 16):
                zb[r, pl.ds(q * 16, 16)] = zeros16
        for k in range(40):
            pltpu.sync_copy(zb, acc.at[pl.ds(s * 640 + k * 16, 16)])
        pltpu.async_copy(srcp_hbm.at[wid, pl.ds(0, _GRP)], sidx.at[0], isem)
        pltpu.async_copy(dstp_hbm.at[wid, pl.ds(0, _GRP)], didx.at[0], isem)
        plsc.subcore_barrier()

        bufs = ((rows0, gs0, ss0), (rows1, gs1, ss1))

        def group(g, carry):
            p = g % 2
            sb = sidx.at[p]
            db = didx.at[p]
            pltpu.make_async_copy(srcp_hbm.at[wid, pl.ds(g * _GRP, _GRP)],
                                  sb, isem).wait()
            pltpu.make_async_copy(dstp_hbm.at[wid, pl.ds(g * _GRP, _GRP)],
                                  db, isem).wait()

            @pl.when(g + 1 < n_grp)
            def _():
                pltpu.async_copy(
                    srcp_hbm.at[wid, pl.ds((g + 1) * _GRP, _GRP)],
                    sidx.at[1 - p], isem)
                pltpu.async_copy(
                    dstp_hbm.at[wid, pl.ds((g + 1) * _GRP, _GRP)],
                    didx.at[1 - p], isem)

            pltpu.async_copy(table_hbm.at[sb.at[0]], rows0, gs0)
            for j in range(_GRP):
                buf, gsem, ssem = bufs[j % 2]
                nbuf, ngsem, nssem = bufs[(j + 1) % 2]
                pltpu.make_async_copy(table_hbm.at[sb.at[j]], buf,
                                      gsem).wait()
                if j >= 1:
                    pltpu.make_async_copy(nbuf, acc.at[db.at[j - 1]],
                                          nssem).wait()
                if j + 1 < _GRP:
                    pltpu.async_copy(table_hbm.at[sb.at[j + 1]], nbuf,
                                     ngsem)
                pltpu.async_copy(buf, acc.at[db.at[j]], ssem, add=True)
            lbuf, _, lssem = bufs[(_GRP - 1) % 2]
            pltpu.make_async_copy(lbuf, acc.at[db.at[_GRP - 1]],
                                  lssem).wait()
            return carry

        lax.fori_loop(0, n_grp, group, 0)
        plsc.subcore_barrier()
        pltpu.sync_copy(acc.at[pl.ds(s * 640, 640)],
                        out_hbm.at[c, pl.ds(s * 640, 640)])

    return agg_kernel(table, srcp, dstp)


def _tc1_body(degT_ref, x_ref, w_ref, out_ref):
    deg = degT_ref[:, 0:1] + degT_ref[:, 1:2] + 1.0
    dinv = lax.rsqrt(deg)
    h = jnp.dot(x_ref[...], w_ref[...], preferred_element_type=jnp.float32)
    out_ref[...] = h * dinv


def _tc_scale_matmul(degT, xp, w):
    return pl.pallas_call(
        _tc1_body,
        grid=(_GRID,),
        in_specs=[
            pl.BlockSpec((_BLK, 2), lambda i: (i, 0)),
            pl.BlockSpec((_BLK, 128), lambda i: (i, 0)),
            pl.BlockSpec((128, 128), lambda i: (0, 0)),
        ],
        out_specs=pl.BlockSpec((_BLK, 128), lambda i: (i, 0)),
        out_shape=jax.ShapeDtypeStruct((_R, 128), jnp.float32),
    )(degT, xp, w)


def _tc2_body(sacc_ref, h1_ref, degT_ref, b1_ref, w2_ref, out_ref):
    deg = degT_ref[:, 0:1] + degT_ref[:, 1:2] + 1.0
    dinv = lax.rsqrt(deg)
    s = sacc_ref[0] + sacc_ref[1] + h1_ref[...]
    z = jnp.maximum(s * dinv + b1_ref[...], 0.0)
    h2 = jnp.dot(z, w2_ref[...], preferred_element_type=jnp.float32)
    out_ref[...] = h2 * dinv


def _tc_mid(sacc, h1, degT, b1, w2):
    return pl.pallas_call(
        _tc2_body,
        grid=(_GRID,),
        in_specs=[
            pl.BlockSpec((2, _BLK, 128), lambda i: (0, i, 0)),
            pl.BlockSpec((_BLK, 128), lambda i: (i, 0)),
            pl.BlockSpec((_BLK, 2), lambda i: (i, 0)),
            pl.BlockSpec((1, 128), lambda i: (0, 0)),
            pl.BlockSpec((128, 64), lambda i: (0, 0)),
        ],
        out_specs=pl.BlockSpec((_BLK, 64), lambda i: (i, 0)),
        out_shape=jax.ShapeDtypeStruct((_R, 64), jnp.float32),
    )(sacc, h1, degT, b1, w2)


def _tc3_body(sacc_ref, h2_ref, degT_ref, b2_ref, out_ref):
    deg = degT_ref[:, 0:1] + degT_ref[:, 1:2] + 1.0
    dinv = lax.rsqrt(deg)
    s = sacc_ref[0] + sacc_ref[1] + h2_ref[...]
    out_ref[...] = s * dinv + b2_ref[...]


def _tc_final(sacc, h2, degT, b2):
    return pl.pallas_call(
        _tc3_body,
        grid=(_GRID,),
        in_specs=[
            pl.BlockSpec((2, _BLK, 64), lambda i: (0, i, 0)),
            pl.BlockSpec((_BLK, 64), lambda i: (i, 0)),
            pl.BlockSpec((_BLK, 2), lambda i: (i, 0)),
            pl.BlockSpec((1, 64), lambda i: (0, 0)),
        ],
        out_specs=pl.BlockSpec((_BLK, 64), lambda i: (i, 0)),
        out_shape=jax.ShapeDtypeStruct((_R, 64), jnp.float32),
    )(sacc, h2, degT, b2)


def kernel(x, edge_index, W1, b1, W2, b2):
    src = edge_index[0]
    dst = edge_index[1]

    w = jnp.arange(_NW, dtype=jnp.int32)[:, None]
    i = jnp.arange(_PAD, dtype=jnp.int32)[None, :]
    pad_src = (w * 997 + i * 131) % _N
    pad_dst = _N + (w + i) % 16
    srcp = jnp.concatenate([src.reshape(_NW, _EPT), pad_src], axis=1)
    dstp = jnp.concatenate([dst.reshape(_NW, _EPT), pad_dst], axis=1)
    srcp = srcp.reshape(_NW, _CPT, _CHUNK)
    dstp = dstp.reshape(_NW, _CPT, _CHUNK)

    degp = _sc_degree(dstp)
    degT = degp.T

    xp = jnp.concatenate(
        [x, jnp.zeros((_R - _N, x.shape[1]), x.dtype)], axis=0)

    h1 = _tc_scale_matmul(degT, xp, W1)
    s1 = _sc_aggregate(h1, srcp, dstp, 128)
    h2 = _tc_mid(s1, h1, degT, b1.reshape(1, -1), W2)
    s2 = _sc_aggregate(h2, srcp, dstp, 64)
    outp = _tc_final(s2, h2, degT, b2.reshape(1, -1))
    return outp[:_N]


# speedup vs baseline: 32.3693x; 1.0698x over previous
import functools

import numpy as np
import jax
import jax.numpy as jnp
from jax import lax
from jax.experimental import pallas as pl
from jax.experimental.pallas import tpu as pltpu
from jax.experimental.pallas import tpu_sc as plsc

_N = 10000
_E = 320000
_R = 10240
_NW = 32
_CHUNK = 128
_CPT = 80
_GRP = 8
_EPT = _E // _NW
_PAD = _CPT * _CHUNK - _EPT
_BLK = 1000
_GRID = _N // _BLK

_PW = np.arange(_NW, dtype=np.int32)[:, None]
_PI = np.arange(_PAD, dtype=np.int32)[None, :]
_PAD_SRC = ((_PW * 997 + _PI * 131) % _N).astype(np.int32)
_PAD_DST = (_N + (_PW + _PI) % 16).astype(np.int32)


def _sc_mesh():
    return plsc.VectorSubcoreMesh(core_axis_name="c", subcore_axis_name="s")


def _sc_degree(dstp):
    @functools.partial(
        pl.kernel,
        out_type=jax.ShapeDtypeStruct((2, _R), jnp.float32),
        mesh=_sc_mesh(),
        scratch_types=[
            pltpu.VMEM((_CPT, _CHUNK), jnp.int32),
            pltpu.VMEM((_CHUNK,), jnp.float32),
            pltpu.VMEM((640,), jnp.float32),
            pltpu.VMEM_SHARED((_R,), jnp.float32),
        ],
    )
    def deg_kernel(dstp_hbm, out_hbm, idx_v, ones_v, zb_v, acc):
        c = lax.axis_index("c")
        s = lax.axis_index("s")
        wid = c * 16 + s
        zeros16 = jnp.zeros((16,), jnp.float32)
        ones16 = jnp.ones((16,), jnp.float32)
        for k in range(40):
            zb_v[pl.ds(k * 16, 16)] = zeros16
        for k in range(8):
            ones_v[pl.ds(k * 16, 16)] = ones16
        pltpu.sync_copy(zb_v, acc.at[pl.ds(s * 640, 640)])
        pltpu.sync_copy(dstp_hbm.at[wid], idx_v)
        plsc.subcore_barrier()

        def body(j, carry):
            pltpu.sync_copy(ones_v, acc.at[idx_v.at[j]], add=True)
            return carry

        lax.fori_loop(0, _CPT, body, 0)
        plsc.subcore_barrier()
        pltpu.sync_copy(acc.at[pl.ds(s * 640, 640)],
                        out_hbm.at[c, pl.ds(s * 640, 640)])

    return deg_kernel(dstp)


def _sc_aggregate(table, srcp, dstp, d, nbuf, tc_tiling):
    scratch = [
        pltpu.VMEM((2, _GRP, _CHUNK), jnp.int32),
        pltpu.VMEM((2, _GRP, _CHUNK), jnp.int32),
        pltpu.VMEM((16, d), jnp.float32),
        pltpu.VMEM_SHARED((_R, d), jnp.float32),
    ]
    scratch += [pltpu.VMEM((_CHUNK, d), jnp.float32)] * nbuf
    scratch += [pltpu.SemaphoreType.DMA] * (2 * nbuf + 1)

    @functools.partial(
        pl.kernel,
        out_type=jax.ShapeDtypeStruct((2, _R, d), jnp.float32),
        mesh=_sc_mesh(),
        compiler_params=pltpu.CompilerParams(use_tc_tiling_on_sc=tc_tiling),
        scratch_types=scratch,
    )
    def agg_kernel(table_hbm, srcp_hbm, dstp_hbm, out_hbm,
                   sidx, didx, zb, acc, *rest):
        rows = rest[:nbuf]
        gs = rest[nbuf:2 * nbuf]
        ss = rest[2 * nbuf:3 * nbuf]
        isem = rest[3 * nbuf]
        c = lax.axis_index("c")
        s = lax.axis_index("s")
        wid = c * 16 + s
        n_grp = _CPT // _GRP
        zeros16 = jnp.zeros((16,), jnp.float32)
        for r in range(16):
            for q in range(d // 16):
                zb[r, pl.ds(q * 16, 16)] = zeros16
        for k in range(40):
            pltpu.sync_copy(zb, acc.at[pl.ds(s * 640 + k * 16, 16)])
        pltpu.async_copy(srcp_hbm.at[wid, pl.ds(0, _GRP)], sidx.at[0], isem)
        pltpu.async_copy(dstp_hbm.at[wid, pl.ds(0, _GRP)], didx.at[0], isem)
        plsc.subcore_barrier()

        def group(g, carry):
            p = g % 2
            sb = sidx.at[p]
            db = didx.at[p]
            pltpu.make_async_copy(srcp_hbm.at[wid, pl.ds(g * _GRP, _GRP)],
                                  sb, isem).wait()
            pltpu.make_async_copy(dstp_hbm.at[wid, pl.ds(g * _GRP, _GRP)],
                                  db, isem).wait()

            @pl.when(g + 1 < n_grp)
            def _():
                pltpu.async_copy(
                    srcp_hbm.at[wid, pl.ds((g + 1) * _GRP, _GRP)],
                    sidx.at[1 - p], isem)
                pltpu.async_copy(
                    dstp_hbm.at[wid, pl.ds((g + 1) * _GRP, _GRP)],
                    didx.at[1 - p], isem)

            for m in range(min(nbuf, _GRP)):
                pltpu.async_copy(table_hbm.at[sb.at[m]], rows[m], gs[m])
            for j in range(_GRP):
                b = j % nbuf
                pltpu.make_async_copy(table_hbm.at[sb.at[j]], rows[b],
                                      gs[b]).wait()
                if j >= 1:
                    pb = (j - 1) % nbuf
                    pltpu.make_async_copy(rows[pb], acc.at[db.at[j - 1]],
                                          ss[pb]).wait()
                    m = j + nbuf - 1
                    if m < _GRP:
                        pltpu.async_copy(table_hbm.at[sb.at[m]], rows[pb],
                                         gs[pb])
                pltpu.async_copy(rows[b], acc.at[db.at[j]], ss[b], add=True)
            lb = (_GRP - 1) % nbuf
            pltpu.make_async_copy(rows[lb], acc.at[db.at[_GRP - 1]],
                                  ss[lb]).wait()
            return carry

        lax.fori_loop(0, n_grp, group, 0)
        plsc.subcore_barrier()
        pltpu.sync_copy(acc.at[pl.ds(s * 640, 640)],
                        out_hbm.at[c, pl.ds(s * 640, 640)])

    return agg_kernel(table, srcp, dstp)


def _tc1_body(degT_ref, x_ref, w_ref, out_ref):
    deg = degT_ref[:, 0:1] + degT_ref[:, 1:2] + 1.0
    dinv = lax.rsqrt(deg)
    h = jnp.dot(x_ref[...], w_ref[...], preferred_element_type=jnp.float32)
    out_ref[...] = h * dinv


def _tc_scale_matmul(degT, x, w):
    return pl.pallas_call(
        _tc1_body,
        grid=(_GRID,),
        in_specs=[
            pl.BlockSpec((_BLK, 2), lambda i: (i, 0)),
            pl.BlockSpec((_BLK, 128), lambda i: (i, 0)),
            pl.BlockSpec((128, 128), lambda i: (0, 0)),
        ],
        out_specs=pl.BlockSpec((_BLK, 128), lambda i: (i, 0)),
        out_shape=jax.ShapeDtypeStruct((_N, 128), jnp.float32),
    )(degT, x, w)


def _tc2_body(sacc_ref, h1_ref, degT_ref, b1_ref, w2_ref, out_ref):
    deg = degT_ref[:, 0:1] + degT_ref[:, 1:2] + 1.0
    dinv = lax.rsqrt(deg)
    s = sacc_ref[0] + sacc_ref[1] + h1_ref[...]
    z = jnp.maximum(s * dinv + b1_ref[...], 0.0)
    h2 = jnp.dot(z, w2_ref[...], preferred_element_type=jnp.float32)
    out_ref[...] = h2 * dinv


def _tc_mid(sacc, h1, degT, b1, w2):
    return pl.pallas_call(
        _tc2_body,
        grid=(_GRID,),
        in_specs=[
            pl.BlockSpec((2, _BLK, 128), lambda i: (0, i, 0)),
            pl.BlockSpec((_BLK, 128), lambda i: (i, 0)),
            pl.BlockSpec((_BLK, 2), lambda i: (i, 0)),
            pl.BlockSpec((1, 128), lambda i: (0, 0)),
            pl.BlockSpec((128, 64), lambda i: (0, 0)),
        ],
        out_specs=pl.BlockSpec((_BLK, 64), lambda i: (i, 0)),
        out_shape=jax.ShapeDtypeStruct((_N, 64), jnp.float32),
    )(sacc, h1, degT, b1, w2)


def _tc3_body(sacc_ref, h2_ref, degT_ref, b2_ref, out_ref):
    deg = degT_ref[:, 0:1] + degT_ref[:, 1:2] + 1.0
    dinv = lax.rsqrt(deg)
    s = sacc_ref[0] + sacc_ref[1] + h2_ref[...]
    out_ref[...] = s * dinv + b2_ref[...]


def _tc_final(sacc, h2, degT, b2):
    return pl.pallas_call(
        _tc3_body,
        grid=(_GRID,),
        in_specs=[
            pl.BlockSpec((2, _BLK, 64), lambda i: (0, i, 0)),
            pl.BlockSpec((_BLK, 64), lambda i: (i, 0)),
            pl.BlockSpec((_BLK, 2), lambda i: (i, 0)),
            pl.BlockSpec((1, 64), lambda i: (0, 0)),
        ],
        out_specs=pl.BlockSpec((_BLK, 64), lambda i: (i, 0)),
        out_shape=jax.ShapeDtypeStruct((_N, 64), jnp.float32),
    )(sacc, h2, degT, b2)


def kernel(x, edge_index, W1, b1, W2, b2):
    src = edge_index[0]
    dst = edge_index[1]

    srcp = jnp.concatenate([src.reshape(_NW, _EPT), _PAD_SRC], axis=1)
    dstp = jnp.concatenate([dst.reshape(_NW, _EPT), _PAD_DST], axis=1)
    srcp = srcp.reshape(_NW, _CPT, _CHUNK)
    dstp = dstp.reshape(_NW, _CPT, _CHUNK)

    degp = _sc_degree(dstp)
    degT = degp.T[:_N]

    h1 = _tc_scale_matmul(degT, x, W1)
    s1 = _sc_aggregate(h1, srcp, dstp, 128, 2, True)
    h2 = _tc_mid(s1, h1, degT, b1.reshape(1, -1), W2)
    s2 = _sc_aggregate(h2, srcp, dstp, 64, 4, False)
    return _tc_final(s2, h2, degT, b2.reshape(1, -1))


# speedup vs baseline: 34.3182x; 1.0602x over previous
import functools

import numpy as np
import jax
import jax.numpy as jnp
from jax import lax
from jax.experimental import pallas as pl
from jax.experimental.pallas import tpu as pltpu
from jax.experimental.pallas import tpu_sc as plsc

_N = 10000
_E = 320000
_R = 10240
_NW = 32
_CHUNK = 128
_CPT = 80
_GRP = 8
_EPT = _E // _NW
_PAD = _CPT * _CHUNK - _EPT
_BLK = 2000
_GRID = _N // _BLK

_NCHUNK = _E // _CHUNK
_PADCHUNK = _NW * _CPT - _NCHUNK
_PI = np.arange(_PADCHUNK * _CHUNK, dtype=np.int32)
_PAD_EDGES = np.stack([
    (_PI * 131 + 17) % _N,
    _N + _PI % 16,
]).reshape(2, _PADCHUNK, _CHUNK).astype(np.int32)


def _sc_mesh():
    return plsc.VectorSubcoreMesh(core_axis_name="c", subcore_axis_name="s")


def _sc_degree(dstp):
    @functools.partial(
        pl.kernel,
        out_type=jax.ShapeDtypeStruct((2, _R), jnp.float32),
        mesh=_sc_mesh(),
        scratch_types=[
            pltpu.VMEM((_CPT, _CHUNK), jnp.int32),
            pltpu.VMEM((_CHUNK,), jnp.float32),
            pltpu.VMEM((640,), jnp.float32),
            pltpu.VMEM_SHARED((_R,), jnp.float32),
        ],
    )
    def deg_kernel(dstp_hbm, out_hbm, idx_v, ones_v, zb_v, acc):
        c = lax.axis_index("c")
        s = lax.axis_index("s")
        wid = c * 16 + s
        zeros16 = jnp.zeros((16,), jnp.float32)
        ones16 = jnp.ones((16,), jnp.float32)
        for k in range(40):
            zb_v[pl.ds(k * 16, 16)] = zeros16
        for k in range(8):
            ones_v[pl.ds(k * 16, 16)] = ones16
        pltpu.sync_copy(zb_v, acc.at[pl.ds(s * 640, 640)])
        pltpu.sync_copy(dstp_hbm.at[wid], idx_v)
        plsc.subcore_barrier()

        def body(j, carry):
            pltpu.sync_copy(ones_v, acc.at[idx_v.at[j]], add=True)
            return carry

        lax.fori_loop(0, _CPT, body, 0)
        plsc.subcore_barrier()
        pltpu.sync_copy(acc.at[pl.ds(s * 640, 640)],
                        out_hbm.at[c, pl.ds(s * 640, 640)])

    return deg_kernel(dstp)


def _sc_aggregate(table, srcp, dstp, d, nbuf, tc_tiling):
    scratch = [
        pltpu.VMEM((2, _GRP, _CHUNK), jnp.int32),
        pltpu.VMEM((2, _GRP, _CHUNK), jnp.int32),
        pltpu.VMEM((16, d), jnp.float32),
        pltpu.VMEM_SHARED((_R, d), jnp.float32),
    ]
    scratch += [pltpu.VMEM((_CHUNK, d), jnp.float32)] * nbuf
    scratch += [pltpu.SemaphoreType.DMA] * (2 * nbuf + 1)

    @functools.partial(
        pl.kernel,
        out_type=jax.ShapeDtypeStruct((2, _R, d), jnp.float32),
        mesh=_sc_mesh(),
        compiler_params=pltpu.CompilerParams(use_tc_tiling_on_sc=tc_tiling),
        scratch_types=scratch,
    )
    def agg_kernel(table_hbm, srcp_hbm, dstp_hbm, out_hbm,
                   sidx, didx, zb, acc, *rest):
        rows = rest[:nbuf]
        gs = rest[nbuf:2 * nbuf]
        ss = rest[2 * nbuf:3 * nbuf]
        isem = rest[3 * nbuf]
        c = lax.axis_index("c")
        s = lax.axis_index("s")
        wid = c * 16 + s
        n_grp = _CPT // _GRP
        zeros16 = jnp.zeros((16,), jnp.float32)
        for r in range(16):
            for q in range(d // 16):
                zb[r, pl.ds(q * 16, 16)] = zeros16
        for k in range(40):
            pltpu.sync_copy(zb, acc.at[pl.ds(s * 640 + k * 16, 16)])
        pltpu.async_copy(srcp_hbm.at[wid, pl.ds(0, _GRP)], sidx.at[0], isem)
        pltpu.async_copy(dstp_hbm.at[wid, pl.ds(0, _GRP)], didx.at[0], isem)
        plsc.subcore_barrier()

        def group(g, carry):
            p = g % 2
            sb = sidx.at[p]
            db = didx.at[p]
            pltpu.make_async_copy(srcp_hbm.at[wid, pl.ds(g * _GRP, _GRP)],
                                  sb, isem).wait()
            pltpu.make_async_copy(dstp_hbm.at[wid, pl.ds(g * _GRP, _GRP)],
                                  db, isem).wait()

            @pl.when(g + 1 < n_grp)
            def _():
                pltpu.async_copy(
                    srcp_hbm.at[wid, pl.ds((g + 1) * _GRP, _GRP)],
                    sidx.at[1 - p], isem)
                pltpu.async_copy(
                    dstp_hbm.at[wid, pl.ds((g + 1) * _GRP, _GRP)],
                    didx.at[1 - p], isem)

            for m in range(min(nbuf, _GRP)):
                pltpu.async_copy(table_hbm.at[sb.at[m]], rows[m], gs[m])
            for j in range(_GRP):
                b = j % nbuf
                pltpu.make_async_copy(table_hbm.at[sb.at[j]], rows[b],
                                      gs[b]).wait()
                if j >= 1:
                    pb = (j - 1) % nbuf
                    pltpu.make_async_copy(rows[pb], acc.at[db.at[j - 1]],
                                          ss[pb]).wait()
                    m = j + nbuf - 1
                    if m < _GRP:
                        pltpu.async_copy(table_hbm.at[sb.at[m]], rows[pb],
                                         gs[pb])
                pltpu.async_copy(rows[b], acc.at[db.at[j]], ss[b], add=True)
            lb = (_GRP - 1) % nbuf
            pltpu.make_async_copy(rows[lb], acc.at[db.at[_GRP - 1]],
                                  ss[lb]).wait()
            return carry

        lax.fori_loop(0, n_grp, group, 0)
        plsc.subcore_barrier()
        pltpu.sync_copy(acc.at[pl.ds(s * 640, 640)],
                        out_hbm.at[c, pl.ds(s * 640, 640)])

    return agg_kernel(table, srcp, dstp)


def _tc1_body(x_ref, w_ref, out_ref):
    out_ref[...] = jnp.dot(x_ref[...], w_ref[...],
                           preferred_element_type=jnp.float32)


def _tc_matmul(x, w):
    return pl.pallas_call(
        _tc1_body,
        grid=(_GRID,),
        in_specs=[
            pl.BlockSpec((_BLK, 128), lambda i: (i, 0)),
            pl.BlockSpec((128, 128), lambda i: (0, 0)),
        ],
        out_specs=pl.BlockSpec((_BLK, 128), lambda i: (i, 0)),
        out_shape=jax.ShapeDtypeStruct((_N, 128), jnp.float32),
    )(x, w)


def _tc_scale_body(degT_ref, h_ref, out_ref):
    deg = degT_ref[:, 0:1] + degT_ref[:, 1:2] + 1.0
    out_ref[...] = h_ref[...] * lax.rsqrt(deg)


def _tc_scale(degT, h):
    return pl.pallas_call(
        _tc_scale_body,
        grid=(_GRID,),
        in_specs=[
            pl.BlockSpec((_BLK, 2), lambda i: (i, 0)),
            pl.BlockSpec((_BLK, 128), lambda i: (i, 0)),
        ],
        out_specs=pl.BlockSpec((_BLK, 128), lambda i: (i, 0)),
        out_shape=jax.ShapeDtypeStruct((_N, 128), jnp.float32),
    )(degT, h)


def _tc2_body(sacc_ref, h1_ref, degT_ref, b1_ref, w2_ref, out_ref):
    deg = degT_ref[:, 0:1] + degT_ref[:, 1:2] + 1.0
    dinv = lax.rsqrt(deg)
    s = sacc_ref[0] + sacc_ref[1] + h1_ref[...]
    z = jnp.maximum(s * dinv + b1_ref[...], 0.0)
    h2 = jnp.dot(z, w2_ref[...], preferred_element_type=jnp.float32)
    out_ref[...] = h2 * dinv


def _tc_mid(sacc, h1, degT, b1, w2):
    return pl.pallas_call(
        _tc2_body,
        grid=(_GRID,),
        in_specs=[
            pl.BlockSpec((2, _BLK, 128), lambda i: (0, i, 0)),
            pl.BlockSpec((_BLK, 128), lambda i: (i, 0)),
            pl.BlockSpec((_BLK, 2), lambda i: (i, 0)),
            pl.BlockSpec((1, 128), lambda i: (0, 0)),
            pl.BlockSpec((128, 64), lambda i: (0, 0)),
        ],
        out_specs=pl.BlockSpec((_BLK, 64), lambda i: (i, 0)),
        out_shape=jax.ShapeDtypeStruct((_N, 64), jnp.float32),
    )(sacc, h1, degT, b1, w2)


def _tc3_body(sacc_ref, h2_ref, degT_ref, b2_ref, out_ref):
    deg = degT_ref[:, 0:1] + degT_ref[:, 1:2] + 1.0
    dinv = lax.rsqrt(deg)
    s = sacc_ref[0] + sacc_ref[1] + h2_ref[...]
    out_ref[...] = s * dinv + b2_ref[...]


def _tc_final(sacc, h2, degT, b2):
    return pl.pallas_call(
        _tc3_body,
        grid=(_GRID,),
        in_specs=[
            pl.BlockSpec((2, _BLK, 64), lambda i: (0, i, 0)),
            pl.BlockSpec((_BLK, 64), lambda i: (i, 0)),
            pl.BlockSpec((_BLK, 2), lambda i: (i, 0)),
            pl.BlockSpec((1, 64), lambda i: (0, 0)),
        ],
        out_specs=pl.BlockSpec((_BLK, 64), lambda i: (i, 0)),
        out_shape=jax.ShapeDtypeStruct((_N, 64), jnp.float32),
    )(sacc, h2, degT, b2)


def kernel(x, edge_index, W1, b1, W2, b2):
    ep = jnp.concatenate(
        [edge_index.reshape(2, _NCHUNK, _CHUNK), _PAD_EDGES], axis=1)
    srcp = ep[0].reshape(_NW, _CPT, _CHUNK)
    dstp = ep[1].reshape(_NW, _CPT, _CHUNK)

    h1r = _tc_matmul(x, W1)
    degp = _sc_degree(dstp)
    degT = degp.T[:_N]

    h1 = _tc_scale(degT, h1r)
    s1 = _sc_aggregate(h1, srcp, dstp, 128, 2, True)
    h2 = _tc_mid(s1, h1, degT, b1.reshape(1, -1), W2)
    s2 = _sc_aggregate(h2, srcp, dstp, 64, 4, False)
    return _tc_final(s2, h2, degT, b2.reshape(1, -1))


# speedup vs baseline: 34.4854x; 1.0049x over previous
import functools

import numpy as np
import jax
import jax.numpy as jnp
from jax import lax
from jax.experimental import pallas as pl
from jax.experimental.pallas import tpu as pltpu
from jax.experimental.pallas import tpu_sc as plsc

_N = 10000
_E = 320000
_R = 10240
_NW = 32
_CHUNK = 128
_CPT = 80
_GRP = 8
_EPT = _E // _NW
_PAD = _CPT * _CHUNK - _EPT
_BLK = 2000
_GRID = _N // _BLK

_NCHUNK = _E // _CHUNK
_PADCHUNK = _NW * _CPT - _NCHUNK
_PI = np.arange(_PADCHUNK * _CHUNK, dtype=np.int32)
_PAD_EDGES = np.stack([
    (_PI * 131 + 17) % _N,
    _N + _PI % 16,
]).reshape(2, _PADCHUNK, _CHUNK).astype(np.int32)


def _sc_mesh():
    return plsc.VectorSubcoreMesh(core_axis_name="c", subcore_axis_name="s")


def _sc_degree(dstp):
    @functools.partial(
        pl.kernel,
        out_type=jax.ShapeDtypeStruct((2, _R), jnp.float32),
        mesh=_sc_mesh(),
        scratch_types=[
            pltpu.VMEM((_CPT, _CHUNK), jnp.int32),
            pltpu.VMEM((_CHUNK,), jnp.float32),
            pltpu.VMEM((640,), jnp.float32),
            pltpu.VMEM_SHARED((_R,), jnp.float32),
        ],
    )
    def deg_kernel(dstp_hbm, out_hbm, idx_v, ones_v, zb_v, acc):
        c = lax.axis_index("c")
        s = lax.axis_index("s")
        wid = c * 16 + s
        zeros16 = jnp.zeros((16,), jnp.float32)
        ones16 = jnp.ones((16,), jnp.float32)
        for k in range(40):
            zb_v[pl.ds(k * 16, 16)] = zeros16
        for k in range(8):
            ones_v[pl.ds(k * 16, 16)] = ones16
        pltpu.sync_copy(zb_v, acc.at[pl.ds(s * 640, 640)])
        pltpu.sync_copy(dstp_hbm.at[wid], idx_v)
        plsc.subcore_barrier()

        def body(j, carry):
            pltpu.sync_copy(ones_v, acc.at[idx_v.at[j]], add=True)
            return carry

        lax.fori_loop(0, _CPT, body, 0)
        plsc.subcore_barrier()
        pltpu.sync_copy(acc.at[pl.ds(s * 640, 640)],
                        out_hbm.at[c, pl.ds(s * 640, 640)])

    return deg_kernel(dstp)


def _sc_aggregate(table, srcp, dstp, d, nbuf, tc_tiling):
    scratch = [
        pltpu.VMEM((2, _GRP, _CHUNK), jnp.int32),
        pltpu.VMEM((2, _GRP, _CHUNK), jnp.int32),
        pltpu.VMEM((16, d), jnp.float32),
        pltpu.VMEM_SHARED((_R, d), jnp.float32),
    ]
    scratch += [pltpu.VMEM((_CHUNK, d), jnp.float32)] * nbuf
    scratch += [pltpu.SemaphoreType.DMA] * (2 * nbuf + 1)

    @functools.partial(
        pl.kernel,
        out_type=jax.ShapeDtypeStruct((2, _R, d), jnp.float32),
        mesh=_sc_mesh(),
        compiler_params=pltpu.CompilerParams(use_tc_tiling_on_sc=tc_tiling),
        scratch_types=scratch,
    )
    def agg_kernel(table_hbm, srcp_hbm, dstp_hbm, out_hbm,
                   sidx, didx, zb, acc, *rest):
        rows = rest[:nbuf]
        gs = rest[nbuf:2 * nbuf]
        ss = rest[2 * nbuf:3 * nbuf]
        isem = rest[3 * nbuf]
        c = lax.axis_index("c")
        s = lax.axis_index("s")
        wid = c * 16 + s
        n_grp = _CPT // _GRP
        zeros16 = jnp.zeros((16,), jnp.float32)
        for r in range(16):
            for q in range(d // 16):
                zb[r, pl.ds(q * 16, 16)] = zeros16
        for k in range(40):
            pltpu.sync_copy(zb, acc.at[pl.ds(s * 640 + k * 16, 16)])
        pltpu.async_copy(srcp_hbm.at[wid, pl.ds(0, _GRP)], sidx.at[0], isem)
        pltpu.async_copy(dstp_hbm.at[wid, pl.ds(0, _GRP)], didx.at[0], isem)
        plsc.subcore_barrier()

        def group(g, carry):
            p = g % 2
            sb = sidx.at[p]
            db = didx.at[p]
            pltpu.make_async_copy(srcp_hbm.at[wid, pl.ds(g * _GRP, _GRP)],
                                  sb, isem).wait()
            pltpu.make_async_copy(dstp_hbm.at[wid, pl.ds(g * _GRP, _GRP)],
                                  db, isem).wait()

            @pl.when(g + 1 < n_grp)
            def _():
                pltpu.async_copy(
                    srcp_hbm.at[wid, pl.ds((g + 1) * _GRP, _GRP)],
                    sidx.at[1 - p], isem)
                pltpu.async_copy(
                    dstp_hbm.at[wid, pl.ds((g + 1) * _GRP, _GRP)],
                    didx.at[1 - p], isem)

            for m in range(min(nbuf, _GRP)):
                pltpu.async_copy(table_hbm.at[sb.at[m]], rows[m], gs[m])
            for j in range(_GRP):
                b = j % nbuf
                pltpu.make_async_copy(table_hbm.at[sb.at[j]], rows[b],
                                      gs[b]).wait()
                if j >= 1:
                    pb = (j - 1) % nbuf
                    pltpu.make_async_copy(rows[pb], acc.at[db.at[j - 1]],
                                          ss[pb]).wait()
                    m = j + nbuf - 1
                    if m < _GRP:
                        pltpu.async_copy(table_hbm.at[sb.at[m]], rows[pb],
                                         gs[pb])
                pltpu.async_copy(rows[b], acc.at[db.at[j]], ss[b], add=True)
            lb = (_GRP - 1) % nbuf
            pltpu.make_async_copy(rows[lb], acc.at[db.at[_GRP - 1]],
                                  ss[lb]).wait()
            return carry

        lax.fori_loop(0, n_grp, group, 0)
        plsc.subcore_barrier()
        pltpu.sync_copy(acc.at[pl.ds(s * 640, 640)],
                        out_hbm.at[c, pl.ds(s * 640, 640)])

    return agg_kernel(table, srcp, dstp)


def _tc1_body(degT_ref, x_ref, w_ref, out_ref):
    deg = degT_ref[:, 0:1] + degT_ref[:, 1:2] + 1.0
    dinv = lax.rsqrt(deg)
    h = jnp.dot(x_ref[...], w_ref[...], preferred_element_type=jnp.float32)
    out_ref[...] = h * dinv


def _tc_scale_matmul(degT, x, w):
    return pl.pallas_call(
        _tc1_body,
        grid=(_GRID,),
        in_specs=[
            pl.BlockSpec((_BLK, 2), lambda i: (i, 0)),
            pl.BlockSpec((_BLK, 128), lambda i: (i, 0)),
            pl.BlockSpec((128, 128), lambda i: (0, 0)),
        ],
        out_specs=pl.BlockSpec((_BLK, 128), lambda i: (i, 0)),
        out_shape=jax.ShapeDtypeStruct((_N, 128), jnp.float32),
    )(degT, x, w)


def _tc2_body(sacc_ref, h1_ref, degT_ref, b1_ref, w2_ref, out_ref):
    deg = degT_ref[:, 0:1] + degT_ref[:, 1:2] + 1.0
    dinv = lax.rsqrt(deg)
    s = sacc_ref[0] + sacc_ref[1] + h1_ref[...]
    z = jnp.maximum(s * dinv + b1_ref[...], 0.0)
    h2 = jnp.dot(z, w2_ref[...], preferred_element_type=jnp.float32)
    out_ref[...] = h2 * dinv


def _tc_mid(sacc, h1, degT, b1, w2):
    return pl.pallas_call(
        _tc2_body,
        grid=(_GRID,),
        in_specs=[
            pl.BlockSpec((2, _BLK, 128), lambda i: (0, i, 0)),
            pl.BlockSpec((_BLK, 128), lambda i: (i, 0)),
            pl.BlockSpec((_BLK, 2), lambda i: (i, 0)),
            pl.BlockSpec((1, 128), lambda i: (0, 0)),
            pl.BlockSpec((128, 64), lambda i: (0, 0)),
        ],
        out_specs=pl.BlockSpec((_BLK, 64), lambda i: (i, 0)),
        out_shape=jax.ShapeDtypeStruct((_N, 64), jnp.float32),
    )(sacc, h1, degT, b1, w2)


def _tc3_body(sacc_ref, h2_ref, degT_ref, b2_ref, out_ref):
    deg = degT_ref[:, 0:1] + degT_ref[:, 1:2] + 1.0
    dinv = lax.rsqrt(deg)
    s = sacc_ref[0] + sacc_ref[1] + h2_ref[...]
    out_ref[...] = s * dinv + b2_ref[...]


def _tc_final(sacc, h2, degT, b2):
    return pl.pallas_call(
        _tc3_body,
        grid=(_GRID,),
        in_specs=[
            pl.BlockSpec((2, _BLK, 64), lambda i: (0, i, 0)),
            pl.BlockSpec((_BLK, 64), lambda i: (i, 0)),
            pl.BlockSpec((_BLK, 2), lambda i: (i, 0)),
            pl.BlockSpec((1, 64), lambda i: (0, 0)),
        ],
        out_specs=pl.BlockSpec((_BLK, 64), lambda i: (i, 0)),
        out_shape=jax.ShapeDtypeStruct((_N, 64), jnp.float32),
    )(sacc, h2, degT, b2)


def kernel(x, edge_index, W1, b1, W2, b2):
    ep = jnp.concatenate(
        [edge_index.reshape(2, _NCHUNK, _CHUNK), _PAD_EDGES], axis=1)
    srcp = ep[0].reshape(_NW, _CPT, _CHUNK)
    dstp = ep[1].reshape(_NW, _CPT, _CHUNK)

    degp = _sc_degree(dstp)
    degT = degp.T

    h1 = _tc_scale_matmul(degT, x, W1)
    s1 = _sc_aggregate(h1, srcp, dstp, 128, 2, True)
    h2 = _tc_mid(s1, h1, degT, b1.reshape(1, -1), W2)
    s2 = _sc_aggregate(h2, srcp, dstp, 64, 4, False)
    return _tc_final(s2, h2, degT, b2.reshape(1, -1))


# speedup vs baseline: 35.7122x; 1.0356x over previous
import functools

import numpy as np
import jax
import jax.numpy as jnp
from jax import lax
from jax.experimental import pallas as pl
from jax.experimental.pallas import tpu as pltpu
from jax.experimental.pallas import tpu_sc as plsc

_N = 10000
_E = 320000
_R = 10240
_NW = 32
_CHUNK = 128
_CPT = 80
_GRP = 8
_EPT = _E // _NW
_PAD = _CPT * _CHUNK - _EPT
_BLK = 2000
_GRID = _N // _BLK

_NCHUNK = _E // _CHUNK
_PADCHUNK = _NW * _CPT - _NCHUNK
_PI = np.arange(_PADCHUNK * _CHUNK, dtype=np.int32)
_PAD_EDGES = np.stack([
    (_PI * 131 + 17) % _N,
    _N + _PI % 16,
]).reshape(2, _PADCHUNK, _CHUNK).astype(np.int32)


def _sc_mesh():
    return plsc.VectorSubcoreMesh(core_axis_name="c", subcore_axis_name="s")


def _sc_degree(dstp):
    @functools.partial(
        pl.kernel,
        out_type=jax.ShapeDtypeStruct((2, _R), jnp.float32),
        mesh=_sc_mesh(),
        scratch_types=[
            pltpu.VMEM((_CPT, _CHUNK), jnp.int32),
            pltpu.VMEM((_CHUNK,), jnp.float32),
            pltpu.VMEM((640,), jnp.float32),
            pltpu.VMEM_SHARED((_R,), jnp.float32),
        ],
    )
    def deg_kernel(dstp_hbm, out_hbm, idx_v, ones_v, zb_v, acc):
        c = lax.axis_index("c")
        s = lax.axis_index("s")
        wid = c * 16 + s
        zeros16 = jnp.zeros((16,), jnp.float32)
        ones16 = jnp.ones((16,), jnp.float32)
        for k in range(40):
            zb_v[pl.ds(k * 16, 16)] = zeros16
        for k in range(8):
            ones_v[pl.ds(k * 16, 16)] = ones16
        pltpu.sync_copy(zb_v, acc.at[pl.ds(s * 640, 640)])
        pltpu.sync_copy(dstp_hbm.at[wid], idx_v)
        plsc.subcore_barrier()

        def body(j, carry):
            pltpu.sync_copy(ones_v, acc.at[idx_v.at[j]], add=True)
            return carry

        lax.fori_loop(0, _CPT, body, 0)
        plsc.subcore_barrier()
        pltpu.sync_copy(acc.at[pl.ds(s * 640, 640)],
                        out_hbm.at[c, pl.ds(s * 640, 640)])

    return deg_kernel(dstp)


def _sc_aggregate(table, srcp, dstp, d, nbuf, tc_tiling):
    scratch = [
        pltpu.VMEM((2, _GRP, _CHUNK), jnp.int32),
        pltpu.VMEM((2, _GRP, _CHUNK), jnp.int32),
        pltpu.VMEM((16, d), jnp.float32),
        pltpu.VMEM_SHARED((_R, d), jnp.float32),
    ]
    scratch += [pltpu.VMEM((_CHUNK, d), jnp.float32)] * nbuf
    scratch += [pltpu.SemaphoreType.DMA] * (2 * nbuf + 1)

    @functools.partial(
        pl.kernel,
        out_type=jax.ShapeDtypeStruct((2, _R, d), jnp.float32),
        mesh=_sc_mesh(),
        compiler_params=pltpu.CompilerParams(use_tc_tiling_on_sc=tc_tiling),
        scratch_types=scratch,
    )
    def agg_kernel(table_hbm, srcp_hbm, dstp_hbm, out_hbm,
                   sidx, didx, zb, acc, *rest):
        rows = rest[:nbuf]
        gs = rest[nbuf:2 * nbuf]
        ss = rest[2 * nbuf:3 * nbuf]
        isem = rest[3 * nbuf]
        c = lax.axis_index("c")
        s = lax.axis_index("s")
        wid = c * 16 + s
        n_grp = _CPT // _GRP
        zeros16 = jnp.zeros((16,), jnp.float32)
        for r in range(16):
            for q in range(d // 16):
                zb[r, pl.ds(q * 16, 16)] = zeros16
        for k in range(40):
            pltpu.sync_copy(zb, acc.at[pl.ds(s * 640 + k * 16, 16)])
        pltpu.async_copy(srcp_hbm.at[wid, pl.ds(0, _GRP)], sidx.at[0], isem)
        pltpu.async_copy(dstp_hbm.at[wid, pl.ds(0, _GRP)], didx.at[0], isem)
        plsc.subcore_barrier()

        def group(g, carry):
            p = g % 2
            sb = sidx.at[p]
            db = didx.at[p]
            pltpu.make_async_copy(srcp_hbm.at[wid, pl.ds(g * _GRP, _GRP)],
                                  sb, isem).wait()
            pltpu.make_async_copy(dstp_hbm.at[wid, pl.ds(g * _GRP, _GRP)],
                                  db, isem).wait()

            @pl.when(g + 1 < n_grp)
            def _():
                pltpu.async_copy(
                    srcp_hbm.at[wid, pl.ds((g + 1) * _GRP, _GRP)],
                    sidx.at[1 - p], isem)
                pltpu.async_copy(
                    dstp_hbm.at[wid, pl.ds((g + 1) * _GRP, _GRP)],
                    didx.at[1 - p], isem)

            for m in range(min(nbuf, _GRP)):
                pltpu.async_copy(table_hbm.at[sb.at[m]], rows[m], gs[m])
            for j in range(_GRP):
                b = j % nbuf
                pltpu.make_async_copy(table_hbm.at[sb.at[j]], rows[b],
                                      gs[b]).wait()
                if j >= 1:
                    pb = (j - 1) % nbuf
                    pltpu.make_async_copy(rows[pb], acc.at[db.at[j - 1]],
                                          ss[pb]).wait()
                    m = j + nbuf - 1
                    if m < _GRP:
                        pltpu.async_copy(table_hbm.at[sb.at[m]], rows[pb],
                                         gs[pb])
                pltpu.async_copy(rows[b], acc.at[db.at[j]], ss[b], add=True)
            lb = (_GRP - 1) % nbuf
            pltpu.make_async_copy(rows[lb], acc.at[db.at[_GRP - 1]],
                                  ss[lb]).wait()
            return carry

        lax.fori_loop(0, n_grp, group, 0)
        plsc.subcore_barrier()
        pltpu.sync_copy(acc.at[pl.ds(s * 640, 640)],
                        out_hbm.at[c, pl.ds(s * 640, 640)])

    return agg_kernel(table, srcp, dstp)


def _tc1_body(degT_ref, x_ref, w_ref, out_ref):
    deg = degT_ref[:, 0:1] + degT_ref[:, 1:2] + 1.0
    dinv = lax.rsqrt(deg)
    h = jnp.dot(x_ref[...], w_ref[...], preferred_element_type=jnp.float32)
    out_ref[...] = h * dinv


def _tc_scale_matmul(degT, x, w):
    return pl.pallas_call(
        _tc1_body,
        grid=(_GRID,),
        in_specs=[
            pl.BlockSpec((_BLK, 2), lambda i: (i, 0)),
            pl.BlockSpec((_BLK, 128), lambda i: (i, 0)),
            pl.BlockSpec((128, 128), lambda i: (0, 0)),
        ],
        out_specs=pl.BlockSpec((_BLK, 128), lambda i: (i, 0)),
        out_shape=jax.ShapeDtypeStruct((_N, 128), jnp.float32),
    )(degT, x, w)


def _tc2_body(sacc_ref, h1_ref, degT_ref, b1_ref, w2_ref, out_ref):
    deg = degT_ref[:, 0:1] + degT_ref[:, 1:2] + 1.0
    dinv = lax.rsqrt(deg)
    s = sacc_ref[0] + sacc_ref[1] + h1_ref[...]
    z = jnp.maximum(s * dinv + b1_ref[...], 0.0)
    h2 = jnp.dot(z, w2_ref[...], preferred_element_type=jnp.float32)
    out_ref[...] = h2 * dinv


def _tc_mid(sacc, h1, degT, b1, w2):
    return pl.pallas_call(
        _tc2_body,
        grid=(_GRID,),
        in_specs=[
            pl.BlockSpec((2, _BLK, 128), lambda i: (0, i, 0)),
            pl.BlockSpec((_BLK, 128), lambda i: (i, 0)),
            pl.BlockSpec((_BLK, 2), lambda i: (i, 0)),
            pl.BlockSpec((1, 128), lambda i: (0, 0)),
            pl.BlockSpec((128, 64), lambda i: (0, 0)),
        ],
        out_specs=pl.BlockSpec((_BLK, 64), lambda i: (i, 0)),
        out_shape=jax.ShapeDtypeStruct((_N, 64), jnp.float32),
    )(sacc, h1, degT, b1, w2)


def _tc3_body(sacc_ref, h2_ref, degq_ref, b2_ref, out_ref):
    dega = degq_ref[:, 0:1] + degq_ref[:, 1:2] + 1.0
    degb = degq_ref[:, 2:3] + degq_ref[:, 3:4] + 1.0
    dinva = lax.rsqrt(dega)
    dinvb = lax.rsqrt(degb)
    blkp = _BLK // 2
    lane = lax.broadcasted_iota(jnp.int32, (blkp, 128), 1)
    dinv = jnp.where(lane < 64, dinva, dinvb)
    s = sacc_ref[0] + sacc_ref[1] + h2_ref[...]
    out_ref[...] = s * dinv + b2_ref[...]


def _tc_final(sacc, h2, degT, b2):
    sacc = sacc.reshape(2, _R // 2, 128)
    h2p = h2.reshape(_N // 2, 128)
    degq = degT.reshape(_R // 2, 4)
    b2p = jnp.concatenate([b2, b2]).reshape(1, 128)
    outp = pl.pallas_call(
        _tc3_body,
        grid=(_GRID,),
        in_specs=[
            pl.BlockSpec((2, _BLK // 2, 128), lambda i: (0, i, 0)),
            pl.BlockSpec((_BLK // 2, 128), lambda i: (i, 0)),
            pl.BlockSpec((_BLK // 2, 4), lambda i: (i, 0)),
            pl.BlockSpec((1, 128), lambda i: (0, 0)),
        ],
        out_specs=pl.BlockSpec((_BLK // 2, 128), lambda i: (i, 0)),
        out_shape=jax.ShapeDtypeStruct((_N // 2, 128), jnp.float32),
    )(sacc, h2p, degq, b2p)
    return outp.reshape(_N, 64)


def kernel(x, edge_index, W1, b1, W2, b2):
    ep = jnp.concatenate(
        [edge_index.reshape(2, _NCHUNK, _CHUNK), _PAD_EDGES], axis=1)
    srcp = ep[0].reshape(_NW, _CPT, _CHUNK)
    dstp = ep[1].reshape(_NW, _CPT, _CHUNK)

    degp = _sc_degree(dstp)
    degT = degp.T

    h1 = _tc_scale_matmul(degT, x, W1)
    s1 = _sc_aggregate(h1, srcp, dstp, 128, 2, True)
    h2 = _tc_mid(s1, h1, degT, b1.reshape(1, -1), W2)
    s2 = _sc_aggregate(h2, srcp, dstp, 64, 4, False)
    return _tc_final(s2, h2, degT, b2.reshape(1, -1))
